# Initial kernel scaffold; baseline (speedup 1.0000x reference)
import jax
import jax.numpy as jnp
from jax.experimental import pallas as pl


def kernel(ui_src, ui_dst, ui_vals, image_feats, text_feats, user_emb, item_emb, W_img, b_img, W_txt, b_txt):
    raise NotImplementedError("write your pallas kernel here")


# speedup vs baseline: 1.3986x; 1.3986x over previous
import functools

import jax
import jax.numpy as jnp
from jax.experimental import pallas as pl
from jax.experimental.pallas import tpu as pltpu

_NU = 50000
_NI = 25000
_D = 64
_IMG = 4096
_TXT = 1024
_CAT = 0.55


def _proj_body(img_ref, txt_ref, iemb_ref, wi_ref, bi_ref, wt_ref, bt_ref, out_ref):
    img = jnp.dot(img_ref[...], wi_ref[...], preferred_element_type=jnp.float32) + bi_ref[...]
    txt = jnp.dot(txt_ref[...], wt_ref[...], preferred_element_type=jnp.float32) + bt_ref[...]
    out_ref[...] = jnp.concatenate([img, txt, iemb_ref[...]], axis=1)


def _project(image_feats, text_feats, item_emb, W_img, b_img, W_txt, b_txt):
    BN = 1000
    grid = (_NI // BN,)
    return pl.pallas_call(
        _proj_body,
        grid=grid,
        in_specs=[
            pl.BlockSpec((BN, _IMG), lambda i: (i, 0)),
            pl.BlockSpec((BN, _TXT), lambda i: (i, 0)),
            pl.BlockSpec((BN, _D), lambda i: (i, 0)),
            pl.BlockSpec((_IMG, _D), lambda i: (0, 0)),
            pl.BlockSpec((1, _D), lambda i: (0, 0)),
            pl.BlockSpec((_TXT, _D), lambda i: (0, 0)),
            pl.BlockSpec((1, _D), lambda i: (0, 0)),
        ],
        out_specs=pl.BlockSpec((BN, 3 * _D), lambda i: (i, 0)),
        out_shape=jax.ShapeDtypeStruct((_NI, 3 * _D), jnp.float32),
    )(image_feats, text_feats, item_emb, W_img, b_img[None, :], W_txt, b_txt[None, :])


def _combine_body(emb_ref, g1_ref, g2_ref, imgf_ref, txtf_ref, out_ref):
    def l2n(x):
        ss = jnp.sum(x * x, axis=1, keepdims=True)
        return x / jnp.sqrt(jnp.maximum(ss, 1e-24))

    mean = (emb_ref[...] + g1_ref[...] + g2_ref[...]) * (1.0 / 3.0)
    out_ref[...] = mean + _CAT * l2n(imgf_ref[...]) + _CAT * l2n(txtf_ref[...])


def _combine(emb, g1, g2, imgf, txtf):
    n = emb.shape[0]
    BN = 1000
    spec = pl.BlockSpec((BN, _D), lambda i: (i, 0))
    return pl.pallas_call(
        _combine_body,
        grid=(n // BN,),
        in_specs=[spec] * 5,
        out_specs=spec,
        out_shape=jax.ShapeDtypeStruct((n, _D), jnp.float32),
    )(emb, g1, g2, imgf, txtf)


def _spmm(rows, cols, vals, x, n):
    return jax.ops.segment_sum(vals[:, None] * jnp.take(x, cols, axis=0), rows, num_segments=n)


def kernel(ui_src, ui_dst, ui_vals, image_feats, text_feats, user_emb, item_emb, W_img, b_img, W_txt, b_txt):
    X1 = _project(image_feats, text_feats, item_emb, W_img, b_img, W_txt, b_txt)
    U1 = _spmm(ui_src, ui_dst, ui_vals, X1, _NU)
    I1 = _spmm(ui_dst, ui_src, ui_vals, U1, _NI)
    img_u, txt_u, u_g1 = U1[:, :64], U1[:, 64:128], U1[:, 128:]
    img_i, txt_i, i_g1 = I1[:, :64], I1[:, 64:128], I1[:, 128:]
    u_g2 = jax.nn.softmax(_spmm(ui_src, ui_dst, ui_vals, i_g1, _NU), axis=-1)
    i_g2 = jax.nn.softmax(_spmm(ui_dst, ui_src, ui_vals, u_g2, _NI), axis=-1)
    u_out = _combine(user_emb, u_g1, u_g2, img_u, txt_u)
    i_out = _combine(item_emb, i_g1, i_g2, img_i, txt_i)
    pu = jnp.zeros((_NU, _D), jnp.float32)
    pi = jnp.zeros((_NI, _D), jnp.float32)
    return (u_out, i_out, img_i, txt_i, img_u, txt_u, u_out, i_out, pu, pi)
